# baseline (speedup 1.0000x reference)
import functools

import jax
import jax.numpy as jnp
from jax import lax
from jax.experimental import pallas as pl
from jax.experimental.pallas import tpu as pltpu
from jax.experimental.pallas import tpu_sc as plsc

N = 10000
E = 160000
D = 256
H = 1024
DH = D // 2
NC = 2
NS = 16
CH = 128
CHUNKS = -(-E // (NS * CH))
EPS = CHUNKS * CH
EPAD = EPS * NS
ACC_ROWS = N + 112
STRIPE = ACC_ROWS // NS


def _sc_aggregate(x2, src, dst, zeros):
    mesh = plsc.VectorSubcoreMesh(core_axis_name="c", subcore_axis_name="s")

    @functools.partial(
        pl.kernel,
        mesh=mesh,
        out_type=jax.ShapeDtypeStruct((NC, ACC_ROWS, DH), jnp.float32),
        scratch_types=[
            pltpu.VMEM((CH,), jnp.int32),
            pltpu.VMEM((CH,), jnp.int32),
            pltpu.VMEM((CH, DH), jnp.float32),
            pltpu.VMEM_SHARED((ACC_ROWS, DH), jnp.float32),
            pltpu.SemaphoreType.DMA,
        ],
    )
    def agg_kernel(x_hbm, src_hbm, dst_hbm, zeros_hbm, out_hbm,
                   src_v, dst_v, rows_v, acc, sem):
        cid = lax.axis_index("c")
        sid = lax.axis_index("s")
        r0 = sid * STRIPE
        pltpu.sync_copy(zeros_hbm, acc.at[pl.ds(r0, STRIPE)])
        plsc.subcore_barrier()

        @pl.loop(0, CHUNKS)
        def _(k):
            base = sid * EPS + k * CH
            pltpu.sync_copy(src_hbm.at[pl.ds(base, CH)], src_v)
            pltpu.sync_copy(dst_hbm.at[pl.ds(base, CH)], dst_v)
            pltpu.async_copy(x_hbm.at[cid].at[src_v], rows_v, sem).wait()
            pltpu.sync_copy(rows_v, acc.at[dst_v], add=True)

        plsc.subcore_barrier()
        pltpu.sync_copy(acc.at[pl.ds(r0, STRIPE)],
                        out_hbm.at[cid, pl.ds(r0, STRIPE)])

    return agg_kernel(x2, src, dst, zeros)


def _mlp_ln(agg, x, W1l, W1h, b1, W2, b2, gamma, beta):
    BR = 1000

    def body(agg_ref, x_ref, w1l_ref, w1h_ref, b1_ref, w2_ref, b2_ref,
             g_ref, bt_ref, o_ref):
        a0 = agg_ref[0].astype(jnp.bfloat16)
        a1 = agg_ref[1].astype(jnp.bfloat16)
        h = jnp.dot(a0, w1l_ref[...], preferred_element_type=jnp.float32)
        h = h + jnp.dot(a1, w1h_ref[...], preferred_element_type=jnp.float32)
        h = jnp.maximum(h + b1_ref[...], 0.0).astype(jnp.bfloat16)
        y = jnp.dot(h, w2_ref[...], preferred_element_type=jnp.float32)
        y = y + b2_ref[...] + x_ref[...]
        mean = jnp.mean(y, axis=-1, keepdims=True)
        c = y - mean
        var = jnp.mean(c * c, axis=-1, keepdims=True)
        o_ref[...] = c * lax.rsqrt(var + 1e-5) * g_ref[...] + bt_ref[...]

    return pl.pallas_call(
        body,
        grid=(N // BR,),
        in_specs=[
            pl.BlockSpec((NC, BR, DH), lambda i: (0, i, 0)),
            pl.BlockSpec((BR, D), lambda i: (i, 0)),
            pl.BlockSpec((DH, H), lambda i: (0, 0)),
            pl.BlockSpec((DH, H), lambda i: (0, 0)),
            pl.BlockSpec((1, H), lambda i: (0, 0)),
            pl.BlockSpec((H, D), lambda i: (0, 0)),
            pl.BlockSpec((1, D), lambda i: (0, 0)),
            pl.BlockSpec((1, D), lambda i: (0, 0)),
            pl.BlockSpec((1, D), lambda i: (0, 0)),
        ],
        out_specs=pl.BlockSpec((BR, D), lambda i: (i, 0)),
        out_shape=jax.ShapeDtypeStruct((N, D), jnp.float32),
        compiler_params=pltpu.CompilerParams(
            dimension_semantics=("parallel",)),
    )(agg, x, W1l, W1h, b1, W2, b2, gamma, beta)


@jax.jit
def kernel(x, edge_index, W1, b1, W2, b2, gamma, beta):
    src = edge_index[0].astype(jnp.int32)
    dst = edge_index[1].astype(jnp.int32)
    src = jnp.pad(src, (0, EPAD - E))
    dst = jnp.pad(dst, (0, EPAD - E), constant_values=N)
    zeros = jnp.zeros((STRIPE, DH), jnp.float32)
    x2 = jnp.stack([x[:, :DH], x[:, DH:]])
    agg = _sc_aggregate(x2, src, dst, zeros)
    W1b = W1.astype(jnp.bfloat16)
    return _mlp_ln(agg, x, W1b[:DH], W1b[DH:], b1.reshape(1, H),
                   W2.astype(jnp.bfloat16), b2.reshape(1, D),
                   gamma.reshape(1, D), beta.reshape(1, D))


# speedup vs baseline: 1.0108x; 1.0108x over previous
import functools

import jax
import jax.numpy as jnp
from jax import lax
from jax.experimental import pallas as pl
from jax.experimental.pallas import tpu as pltpu
from jax.experimental.pallas import tpu_sc as plsc

N = 10000
E = 160000
D = 256
H = 1024
DH = D // 2
NC = 2
NS = 16
CH = 128
CHUNKS = -(-E // (NS * CH))
EPS = CHUNKS * CH
EPAD = EPS * NS
ACC_ROWS = N + 112
STRIPE = ACC_ROWS // NS


def _sc_aggregate(x, src, dst, zeros):
    mesh = plsc.VectorSubcoreMesh(core_axis_name="c", subcore_axis_name="s")

    @functools.partial(
        pl.kernel,
        mesh=mesh,
        out_type=jax.ShapeDtypeStruct((NC, ACC_ROWS, DH), jnp.float32),
        scratch_types=[
            pltpu.VMEM((CH,), jnp.int32),
            pltpu.VMEM((CH,), jnp.int32),
            pltpu.VMEM((CH, DH), jnp.float32),
            pltpu.VMEM_SHARED((ACC_ROWS, DH), jnp.float32),
            pltpu.SemaphoreType.DMA,
        ],
    )
    def agg_kernel(x_hbm, src_hbm, dst_hbm, zeros_hbm, out_hbm,
                   src_v, dst_v, rows_v, acc, sem):
        cid = lax.axis_index("c")
        sid = lax.axis_index("s")
        r0 = sid * STRIPE
        c0 = cid * DH

        def gather():
            return pltpu.make_async_copy(x_hbm.at[src_v, pl.ds(c0, DH)],
                                         rows_v, sem)

        pltpu.sync_copy(src_hbm.at[pl.ds(sid * EPS, CH)], src_v)
        pltpu.sync_copy(dst_hbm.at[pl.ds(sid * EPS, CH)], dst_v)
        gather().start()
        pltpu.sync_copy(zeros_hbm, acc.at[pl.ds(r0, STRIPE)])
        plsc.subcore_barrier()

        @pl.loop(0, CHUNKS)
        def _(k):
            gather().wait()
            pltpu.sync_copy(rows_v, acc.at[dst_v], add=True)

            @pl.when(k < CHUNKS - 1)
            def _():
                base = sid * EPS + (k + 1) * CH
                pltpu.sync_copy(src_hbm.at[pl.ds(base, CH)], src_v)
                pltpu.sync_copy(dst_hbm.at[pl.ds(base, CH)], dst_v)
                gather().start()

        plsc.subcore_barrier()
        pltpu.sync_copy(acc.at[pl.ds(r0, STRIPE)],
                        out_hbm.at[cid, pl.ds(r0, STRIPE)])

    return agg_kernel(x, src, dst, zeros)


def _mlp_ln(agg, x, W1l, W1h, b1, W2, b2, gamma, beta):
    BR = 1000

    def body(agg_ref, x_ref, w1l_ref, w1h_ref, b1_ref, w2_ref, b2_ref,
             g_ref, bt_ref, o_ref):
        a0 = agg_ref[0].astype(jnp.bfloat16)
        a1 = agg_ref[1].astype(jnp.bfloat16)
        h = jnp.dot(a0, w1l_ref[...], preferred_element_type=jnp.float32)
        h = h + jnp.dot(a1, w1h_ref[...], preferred_element_type=jnp.float32)
        h = jnp.maximum(h + b1_ref[...], 0.0).astype(jnp.bfloat16)
        y = jnp.dot(h, w2_ref[...], preferred_element_type=jnp.float32)
        y = y + b2_ref[...] + x_ref[...]
        mean = jnp.mean(y, axis=-1, keepdims=True)
        c = y - mean
        var = jnp.mean(c * c, axis=-1, keepdims=True)
        o_ref[...] = c * lax.rsqrt(var + 1e-5) * g_ref[...] + bt_ref[...]

    return pl.pallas_call(
        body,
        grid=(N // BR,),
        in_specs=[
            pl.BlockSpec((NC, BR, DH), lambda i: (0, i, 0)),
            pl.BlockSpec((BR, D), lambda i: (i, 0)),
            pl.BlockSpec((DH, H), lambda i: (0, 0)),
            pl.BlockSpec((DH, H), lambda i: (0, 0)),
            pl.BlockSpec((1, H), lambda i: (0, 0)),
            pl.BlockSpec((H, D), lambda i: (0, 0)),
            pl.BlockSpec((1, D), lambda i: (0, 0)),
            pl.BlockSpec((1, D), lambda i: (0, 0)),
            pl.BlockSpec((1, D), lambda i: (0, 0)),
        ],
        out_specs=pl.BlockSpec((BR, D), lambda i: (i, 0)),
        out_shape=jax.ShapeDtypeStruct((N, D), jnp.float32),
        compiler_params=pltpu.CompilerParams(
            dimension_semantics=("parallel",)),
    )(agg, x, W1l, W1h, b1, W2, b2, gamma, beta)


@jax.jit
def kernel(x, edge_index, W1, b1, W2, b2, gamma, beta):
    src = edge_index[0].astype(jnp.int32)
    dst = edge_index[1].astype(jnp.int32)
    src = jnp.pad(src, (0, EPAD - E))
    dst = jnp.pad(dst, (0, EPAD - E), constant_values=N)
    zeros = jnp.zeros((STRIPE, DH), jnp.float32)
    agg = _sc_aggregate(x, src, dst, zeros)
    W1b = W1.astype(jnp.bfloat16)
    return _mlp_ln(agg, x, W1b[:DH], W1b[DH:], b1.reshape(1, H),
                   W2.astype(jnp.bfloat16), b2.reshape(1, D),
                   gamma.reshape(1, D), beta.reshape(1, D))
